# baseline (speedup 1.0000x reference)
import numpy as np
import jax
import jax.numpy as jnp
from jax import lax
from jax.experimental import pallas as pl
from jax.experimental.pallas import tpu as pltpu
from jax.experimental.pallas import tpu_sc as plsc

N = 64
D = N * (N + 1) // 2
NN = N * N
L = 16
NC, NS = 2, 16
NW = NC * NS
KX = 8
KO = 2
NOB = 3


def _gather_map() -> np.ndarray:
    iu, ju = np.triu_indices(N)
    M = np.zeros((N, N), np.int32)
    M[iu, ju] = np.arange(iu.size, dtype=np.int32)
    M = M + M.T - np.diag(np.diag(M))
    return M.reshape(-1)


def _body(x_hbm, gmap_hbm, out_hbm, gmap_v, x_v0, x_v1, o_v0, o_v1, o_v2,
          sx0, sx1, so0, so1, so2):
    cid = lax.axis_index("c")
    sid = lax.axis_index("s")
    wid = sid * NC + cid
    rows_per_w = x_hbm.shape[0] // NW
    nchunks = rows_per_w // KX
    base = wid * rows_per_w

    x_bufs = (x_v0, x_v1)
    o_bufs = (o_v0, o_v1, o_v2)
    sx = (sx0, sx1)
    so = (so0, so1, so2)

    pltpu.sync_copy(gmap_hbm, gmap_v)

    def start_in(g, s):
        pltpu.async_copy(
            x_hbm.at[pl.ds(base + g * KX, KX)], x_bufs[s], sx[s]
        )

    start_in(0, 0)
    start_in(1, 1)

    @pl.loop(0, nchunks, step=2)
    def _chunk(g0):
        for s in range(2):
            g = g0 + s
            x_v = x_bufs[s]
            pltpu.make_async_copy(
                x_hbm.at[pl.ds(0, KX)], x_v, sx[s]
            ).wait()

            for h in range(KX // KO):
                jmod = (g * (KX // KO) + h) % NOB
                o_v0_, o_v1_, o_v2_ = o_bufs

                def sub(o_v, sem, first_round):
                    @pl.when(jnp.logical_not(first_round))
                    def _():
                        pltpu.make_async_copy(
                            o_v, out_hbm.at[pl.ds(0, KO)], sem
                        ).wait()

                    @plsc.parallel_loop(0, N, unroll=4)
                    def _i(i):
                        for k in range(N // L):
                            idx = gmap_v[pl.ds(i * N + k * L, L)]
                            for b in range(KO):
                                vals = plsc.load_gather(
                                    x_v,
                                    [jnp.full((L,), h * KO + b, jnp.int32),
                                     idx],
                                )
                                o_v[b, i, pl.ds(k * L, L)] = vals

                    pltpu.async_copy(
                        o_v,
                        out_hbm.at[pl.ds(base + g * KX + h * KO, KO)],
                        sem,
                    )

                first = (g * (KX // KO) + h) < NOB
                for r in range(NOB):
                    @pl.when(jmod == r)
                    def _(r=r):
                        sub(o_bufs[r], so[r], first)

            @pl.when(g + 2 < nchunks)
            def _():
                start_in(g + 2, s)

    for h in range(NOB):
        pltpu.make_async_copy(
            o_bufs[h], out_hbm.at[pl.ds(0, KO)], so[h]
        ).wait()


def kernel(input):
    B = input.shape[0]
    gmap = jnp.asarray(_gather_map())
    mesh = plsc.VectorSubcoreMesh(
        core_axis_name="c", subcore_axis_name="s", num_cores=NC, num_subcores=NS
    )
    run = pl.kernel(
        _body,
        out_type=jax.ShapeDtypeStruct((B, N, N), jnp.float32),
        mesh=mesh,
        scratch_types=[
            pltpu.VMEM((NN,), jnp.int32),
            pltpu.VMEM((KX, D), jnp.float32),
            pltpu.VMEM((KX, D), jnp.float32),
            pltpu.VMEM((KO, N, N), jnp.float32),
            pltpu.VMEM((KO, N, N), jnp.float32),
            pltpu.VMEM((KO, N, N), jnp.float32),
            pltpu.SemaphoreType.DMA,
            pltpu.SemaphoreType.DMA,
            pltpu.SemaphoreType.DMA,
            pltpu.SemaphoreType.DMA,
            pltpu.SemaphoreType.DMA,
        ],
        compiler_params=pltpu.CompilerParams(
            use_tc_tiling_on_sc=True, needs_layout_passes=False
        ),
    )
    return run(input, gmap)


# speedup vs baseline: 1.0215x; 1.0215x over previous
import numpy as np
import jax
import jax.numpy as jnp
from jax import lax
from jax.experimental import pallas as pl
from jax.experimental.pallas import tpu as pltpu
from jax.experimental.pallas import tpu_sc as plsc

N = 64
D = N * (N + 1) // 2
NN = N * N
L = 16
NC, NS = 2, 16
NW = NC * NS
KB = 4


def _gather_map() -> np.ndarray:
    iu, ju = np.triu_indices(N)
    M = np.zeros((N, N), np.int32)
    M[iu, ju] = np.arange(iu.size, dtype=np.int32)
    M = M + M.T - np.diag(np.diag(M))
    return M.reshape(-1)


def _body(x_hbm, gmap_hbm, out_hbm, gmap_v, x_v0, x_v1, o_v0, o_v1,
          sx0, sx1, so0, so1):
    cid = lax.axis_index("c")
    sid = lax.axis_index("s")
    wid = sid * NC + cid
    rows_per_w = x_hbm.shape[0] // NW
    nchunks = rows_per_w // KB
    base = wid * rows_per_w

    x_bufs = (x_v0, x_v1)
    o_bufs = (o_v0, o_v1)
    sx = (sx0, sx1)
    so = (so0, so1)

    pltpu.sync_copy(gmap_hbm, gmap_v)

    def start_in(g, s):
        pltpu.async_copy(
            x_hbm.at[pl.ds(base + g * KB, KB)], x_bufs[s], sx[s]
        )

    start_in(0, 0)
    start_in(1, 1)

    @pl.loop(0, nchunks, step=2)
    def _chunk(g0):
        for s in range(2):
            g = g0 + s
            x_v = x_bufs[s]
            o_v = o_bufs[s]
            pltpu.make_async_copy(
                x_hbm.at[pl.ds(0, KB)], x_v, sx[s]
            ).wait()

            @pl.when(g >= 2)
            def _():
                pltpu.make_async_copy(
                    o_v, out_hbm.at[pl.ds(0, KB)], so[s]
                ).wait()

            @plsc.parallel_loop(0, N, unroll=2)
            def _i(i):
                for k in range(N // L):
                    idx = gmap_v[pl.ds(i * N + k * L, L)]
                    for b in range(KB):
                        vals = plsc.load_gather(
                            x_v, [jnp.full((L,), b, jnp.int32), idx]
                        )
                        o_v[b, i, pl.ds(k * L, L)] = vals

            pltpu.async_copy(
                o_v, out_hbm.at[pl.ds(base + g * KB, KB)], so[s]
            )

            @pl.when(g + 2 < nchunks)
            def _():
                start_in(g + 2, s)

    for s in range(2):
        pltpu.make_async_copy(
            o_bufs[s], out_hbm.at[pl.ds(0, KB)], so[s]
        ).wait()


def kernel(input):
    B = input.shape[0]
    gmap = jnp.asarray(_gather_map())
    mesh = plsc.VectorSubcoreMesh(
        core_axis_name="c", subcore_axis_name="s", num_cores=NC, num_subcores=NS
    )
    run = pl.kernel(
        _body,
        out_type=jax.ShapeDtypeStruct((B, N, N), jnp.float32),
        mesh=mesh,
        scratch_types=[
            pltpu.VMEM((NN,), jnp.int32),
            pltpu.VMEM((KB, D), jnp.float32),
            pltpu.VMEM((KB, D), jnp.float32),
            pltpu.VMEM((KB, N, N), jnp.float32),
            pltpu.VMEM((KB, N, N), jnp.float32),
            pltpu.SemaphoreType.DMA,
            pltpu.SemaphoreType.DMA,
            pltpu.SemaphoreType.DMA,
            pltpu.SemaphoreType.DMA,
        ],
        compiler_params=pltpu.CompilerParams(
            use_tc_tiling_on_sc=True, needs_layout_passes=False
        ),
    )
    return run(input, gmap)
